# baseline (speedup 1.0000x reference)
import functools

import jax
import jax.numpy as jnp
from jax import lax
from jax.experimental import pallas as pl
from jax.experimental.pallas import tpu as pltpu
from jax.experimental.pallas import tpu_sc as plsc

N = 10000
D = 256
DH = 128
E = 160000
NCORE = 2
NSUB = 16
NBUF = 3
EPT = E // NSUB
K = 96
NCHUNK = 108
EPTP = NCHUNK * K
NP = 10240
DUMMY = NP - 4
RPT = NP // NSUB
ROWB = 1000


def _mm_body(x_ref, w_ref, e_ref, o_ref, p_ref):
    o_ref[0] = lax.dot_general(
        x_ref[...], w_ref[...], (((1,), (1,)), ((), ())),
        preferred_element_type=jnp.float32)
    row = jnp.concatenate(
        [e_ref[0, 0, 0], jnp.full((EPTP - EPT,), DUMMY, jnp.int32)])
    col = jnp.concatenate(
        [e_ref[1, 0, 0], jnp.zeros((EPTP - EPT,), jnp.int32)])
    hi = row << 15
    p_ref[0, 0] = col | hi
    p_ref[0, 1] = (col + N) | hi


def _xw_split(x, W, e):
    return pl.pallas_call(
        _mm_body,
        grid=(NCORE, N // ROWB),
        in_specs=[pl.BlockSpec((ROWB, D), lambda c, r: (r, 0)),
                  pl.BlockSpec((DH, D), lambda c, r: (c, 0)),
                  pl.BlockSpec((2, 1, 1, EPT),
                               lambda c, r: (0, jnp.minimum(c * 8 + r, 15),
                                             0, 0))],
        out_specs=[pl.BlockSpec((1, ROWB, DH), lambda c, r: (c, r, 0)),
                   pl.BlockSpec((1, 2, EPTP),
                                lambda c, r: (jnp.minimum(c * 8 + r, 15), 0, 0))],
        out_shape=[jax.ShapeDtypeStruct((NCORE, N, DH), jnp.float32),
                   jax.ShapeDtypeStruct((NSUB, 2, EPTP), jnp.int32)],
    )(x, W, e)


def _sc_scatter(xw_flat, packed_idx):
    mesh = plsc.VectorSubcoreMesh(core_axis_name="c", subcore_axis_name="s")

    @functools.partial(
        pl.kernel,
        out_type=[jax.ShapeDtypeStruct((NCORE, NP, DH), jnp.float32),
                  jax.ShapeDtypeStruct((NCORE, NP), jnp.float32)],
        mesh=mesh,
        scratch_types=[
            pltpu.VMEM_SHARED((NP, DH), jnp.float32),
            pltpu.VMEM_SHARED((NP,), jnp.float32),
            pltpu.VMEM((EPTP,), jnp.int32),
            pltpu.VMEM((NBUF, K, DH), jnp.float32),
            pltpu.VMEM((K,), jnp.int32),
            pltpu.VMEM((K,), jnp.int32),
            pltpu.VMEM((K,), jnp.int32),
            pltpu.VMEM((K,), jnp.int32),
            pltpu.VMEM((K,), jnp.int32),
            pltpu.VMEM((K,), jnp.int32),
            pltpu.VMEM((K,), jnp.float32),
            pltpu.SemaphoreType.DMA,
            pltpu.SemaphoreType.DMA,
            pltpu.SemaphoreType.DMA,
            pltpu.SemaphoreType.DMA,
            pltpu.SemaphoreType.DMA,
            pltpu.SemaphoreType.DMA,
        ],
    )
    def k(xw_hbm, pidx_hbm, out_hbm, deg_hbm, acc, dacc, pbuf, gbuf,
          cb0, rb0, cb1, rb1, cb2, rb2, obuf,
          sg0, sg1, sg2, ss0, ss1, ss2):
        c = lax.axis_index("c")
        s = lax.axis_index("s")
        r0 = s * RPT
        cbs = [cb0, cb1, cb2]
        rbs = [rb0, rb1, rb2]
        sgs = [sg0, sg1, sg2]
        sss = [ss0, ss1, ss2]

        pltpu.sync_copy(pidx_hbm.at[s, c], pbuf)

        def unpack(j, q):
            for t in range(K // 16):
                p = pbuf[pl.ds(j * K + t * 16, 16)]
                sl = pl.ds(t * 16, 16)
                cbs[q][sl] = p & 0x7FFF
                rbs[q][sl] = lax.shift_right_logical(p, 15)

        unpack(0, 0)
        pltpu.make_async_copy(xw_hbm.at[cb0], gbuf.at[0], sg0).start()

        z16 = jnp.zeros((16,), jnp.float32)

        def zfill(j, carry):
            gbuf[2, j // 8, pl.ds((j % 8) * 16, 16)] = z16
            return carry

        lax.fori_loop(0, K * 8, zfill, 0)
        for j in range(K // 16):
            obuf[pl.ds(j * 16, 16)] = z16
        for t in range(6):
            pltpu.sync_copy(gbuf.at[2], acc.at[pl.ds(r0 + t * K, K)])
        pltpu.sync_copy(gbuf.at[2, pl.ds(0, 64)],
                        acc.at[pl.ds(r0 + 6 * K, 64)])
        for t in range(6):
            pltpu.sync_copy(obuf, dacc.at[pl.ds(r0 + t * K, K)])
        pltpu.sync_copy(obuf.at[pl.ds(0, 64)],
                        dacc.at[pl.ds(r0 + 6 * K, 64)])
        o16 = jnp.ones((16,), jnp.float32)
        for j in range(K // 16):
            obuf[pl.ds(j * 16, 16)] = o16
        unpack(1, 1)
        unpack(2, 2)
        plsc.subcore_barrier()
        pltpu.make_async_copy(xw_hbm.at[cb1], gbuf.at[1], sg1).start()
        pltpu.make_async_copy(xw_hbm.at[cb2], gbuf.at[2], sg2).start()

        def body(i, carry):
            for q in range(NBUF):
                pltpu.make_async_copy(xw_hbm.at[cbs[q]], gbuf.at[q],
                                      sgs[q]).wait()
                pltpu.make_async_copy(gbuf.at[q], acc.at[rbs[q]],
                                      sss[q]).start(add=True)

                @pl.when(c == lax.rem(i + q, 2))
                def _():
                    pltpu.sync_copy(obuf, dacc.at[rbs[q]], add=True)

            for q in range(NBUF):
                pltpu.make_async_copy(gbuf.at[q], acc.at[rbs[q]],
                                      sss[q]).wait()
                unpack(NBUF * i + NBUF + q, q)
                pltpu.make_async_copy(xw_hbm.at[cbs[q]], gbuf.at[q],
                                      sgs[q]).start()
            return carry

        lax.fori_loop(0, NCHUNK // NBUF - 1, body, 0)
        for q in range(NBUF):
            pltpu.make_async_copy(xw_hbm.at[cbs[q]], gbuf.at[q],
                                  sgs[q]).wait()
            pltpu.make_async_copy(gbuf.at[q], acc.at[rbs[q]],
                                  sss[q]).start(add=True)

            @pl.when(c == (NCHUNK // NBUF - 1 + q) % 2)
            def _():
                pltpu.sync_copy(obuf, dacc.at[rbs[q]], add=True)

        for q in range(NBUF):
            pltpu.make_async_copy(gbuf.at[q], acc.at[rbs[q]], sss[q]).wait()
        plsc.subcore_barrier()
        for t in range(6):
            sl = pl.ds(r0 + t * K, K)
            pltpu.sync_copy(acc.at[sl], out_hbm.at[c].at[sl])
        sl = pl.ds(r0 + 6 * K, 64)
        pltpu.sync_copy(acc.at[sl], out_hbm.at[c].at[sl])
        pltpu.sync_copy(dacc.at[pl.ds(r0, RPT)],
                        deg_hbm.at[c].at[pl.ds(r0, RPT)])

    return k(xw_flat, packed_idx)


def _finish_body(a_ref, d0_ref, d1_ref, b_ref, o_ref):
    d = jnp.maximum(d0_ref[...] + d1_ref[...], 1.0)
    o_ref[:, :DH] = a_ref[0] / d + b_ref[0, :DH]
    o_ref[:, DH:] = a_ref[1] / d + b_ref[0, DH:]


def _finish(acc2, dega, degb, b2):
    return pl.pallas_call(
        _finish_body,
        grid=(N // ROWB,),
        in_specs=[pl.BlockSpec((NCORE, ROWB, DH), lambda r: (0, r, 0)),
                  pl.BlockSpec((ROWB, 1), lambda r: (r, 0)),
                  pl.BlockSpec((ROWB, 1), lambda r: (r, 0)),
                  pl.BlockSpec((1, D), lambda r: (0, 0))],
        out_specs=pl.BlockSpec((ROWB, D), lambda r: (r, 0)),
        out_shape=jax.ShapeDtypeStruct((N, D), jnp.float32),
    )(acc2, dega, degb, b2)


def kernel(x, edge_index, W, b):
    e4 = edge_index.astype(jnp.int32).reshape(2, NSUB, 1, EPT)
    xw, packed = _xw_split(x, W, e4)
    xw_flat = xw.reshape(NCORE * N, DH)
    acc, deg = _sc_scatter(xw_flat, packed)
    return _finish(acc, deg[0, :N].reshape(N, 1), deg[1, :N].reshape(N, 1),
                   b.reshape(1, D))


# speedup vs baseline: 1.7202x; 1.7202x over previous
import functools

import jax
import jax.numpy as jnp
from jax import lax
from jax.experimental import pallas as pl
from jax.experimental.pallas import tpu as pltpu
from jax.experimental.pallas import tpu_sc as plsc

N = 10000
D = 256
DH = 128
E = 160000
NCORE = 2
NSUB = 16
EPT = E // NSUB
K = 128
NCHUNK = 79
EPTP = NCHUNK * K
NP = 10240
DUMMY = NP - 4
RPT = NP // NSUB
RCHUNK = 128
NRC = RPT // RCHUNK
ROWB = 1000


def _mm_body(x_ref, w_ref, e_ref, o_ref, p_ref):
    o_ref[0] = lax.dot_general(
        x_ref[...], w_ref[...], (((1,), (1,)), ((), ())),
        preferred_element_type=jnp.float32)
    row = jnp.concatenate(
        [e_ref[0, 0, 0], jnp.full((EPTP - EPT,), DUMMY, jnp.int32)])
    col = jnp.concatenate(
        [e_ref[1, 0, 0], jnp.zeros((EPTP - EPT,), jnp.int32)])
    hi = row << 15
    p_ref[0, 0] = col | hi
    p_ref[0, 1] = (col + N) | hi


def _xw_split(x, W, e):
    return pl.pallas_call(
        _mm_body,
        grid=(NCORE, N // ROWB),
        in_specs=[pl.BlockSpec((ROWB, D), lambda c, r: (r, 0)),
                  pl.BlockSpec((DH, D), lambda c, r: (c, 0)),
                  pl.BlockSpec((2, 1, 1, EPT),
                               lambda c, r: (0, jnp.minimum(c * 8 + r, 15),
                                             0, 0))],
        out_specs=[pl.BlockSpec((1, ROWB, DH), lambda c, r: (c, r, 0)),
                   pl.BlockSpec((1, 2, EPTP),
                                lambda c, r: (jnp.minimum(c * 8 + r, 15), 0, 0))],
        out_shape=[jax.ShapeDtypeStruct((NCORE, N, DH), jnp.float32),
                   jax.ShapeDtypeStruct((NSUB, 2, EPTP), jnp.int32)],
    )(x, W, e)


def _sc_scatter(xw_flat, packed_idx):
    mesh = plsc.VectorSubcoreMesh(core_axis_name="c", subcore_axis_name="s")

    @functools.partial(
        pl.kernel,
        out_type=[jax.ShapeDtypeStruct((NCORE, NP, DH), jnp.float32),
                  jax.ShapeDtypeStruct((NCORE, NP), jnp.float32)],
        mesh=mesh,
        scratch_types=[
            pltpu.VMEM_SHARED((NP, DH), jnp.float32),
            pltpu.VMEM_SHARED((NP,), jnp.float32),
            pltpu.VMEM((NCHUNK, K), jnp.int32),
            pltpu.VMEM((K, DH), jnp.float32),
            pltpu.VMEM((K, DH), jnp.float32),
            pltpu.VMEM((K,), jnp.int32),
            pltpu.VMEM((K,), jnp.int32),
            pltpu.VMEM((K,), jnp.int32),
            pltpu.VMEM((K,), jnp.int32),
            pltpu.VMEM((K,), jnp.float32),
            pltpu.SemaphoreType.DMA,
            pltpu.SemaphoreType.DMA,
        ],
    )
    def k(xw_hbm, pidx_hbm, out_hbm, deg_hbm, acc, dacc, pbuf,
          gbufA, gbufB, cbA, rbA, cbB, rbB, obuf, semA, semB):
        c = lax.axis_index("c")
        s = lax.axis_index("s")
        r0 = s * RPT
        is0 = c == 0

        pltpu.sync_copy(pidx_hbm.at[s, c], pbuf)

        def unpack(j, cb, rb):
            for t in range(K // 16):
                sl = pl.ds(t * 16, 16)
                p = pbuf[j, sl]
                cb[sl] = p & 0x7FFF
                rb[sl] = lax.shift_right_logical(p, 15)

        unpack(0, cbA, rbA)
        pltpu.make_async_copy(xw_hbm.at[cbA], gbufA, semA).start()

        z16 = jnp.zeros((16,), jnp.float32)

        def zfill(j, carry):
            gbufB[j // 8, pl.ds((j % 8) * 16, 16)] = z16
            return carry

        lax.fori_loop(0, K * 8, zfill, 0)
        for j in range(K // 16):
            obuf[pl.ds(j * 16, 16)] = z16
        for t in range(NRC):
            pltpu.sync_copy(gbufB, acc.at[pl.ds(r0 + t * RCHUNK, RCHUNK)])
        for t in range(NRC):
            pltpu.sync_copy(obuf, dacc.at[pl.ds(r0 + t * RCHUNK, RCHUNK)])
        o16 = jnp.ones((16,), jnp.float32)
        for j in range(K // 16):
            obuf[pl.ds(j * 16, 16)] = o16
        unpack(1, cbB, rbB)
        plsc.subcore_barrier()

        def body(i, carry):
            j0 = 2 * i
            pltpu.make_async_copy(xw_hbm.at[cbB], gbufB, semB).start()
            pltpu.make_async_copy(xw_hbm.at[cbA], gbufA, semA).wait()
            pltpu.make_async_copy(gbufA, acc.at[rbA], semA).start(add=True)

            @pl.when(is0)
            def _():
                pltpu.sync_copy(obuf, dacc.at[rbA], add=True)

            pltpu.make_async_copy(gbufA, acc.at[rbA], semA).wait()
            unpack(j0 + 2, cbA, rbA)
            pltpu.make_async_copy(xw_hbm.at[cbA], gbufA, semA).start()
            pltpu.make_async_copy(xw_hbm.at[cbB], gbufB, semB).wait()
            pltpu.make_async_copy(gbufB, acc.at[rbB], semB).start(add=True)

            @pl.when(jnp.logical_not(is0))
            def _():
                pltpu.sync_copy(obuf, dacc.at[rbB], add=True)

            pltpu.make_async_copy(gbufB, acc.at[rbB], semB).wait()

            @pl.when(j0 + 3 < NCHUNK)
            def _():
                unpack(j0 + 3, cbB, rbB)

            return carry

        lax.fori_loop(0, (NCHUNK - 1) // 2, body, 0)
        pltpu.make_async_copy(xw_hbm.at[cbA], gbufA, semA).wait()
        pltpu.sync_copy(gbufA, acc.at[rbA], add=True)

        @pl.when(is0)
        def _():
            pltpu.sync_copy(obuf, dacc.at[rbA], add=True)

        plsc.subcore_barrier()
        for t in range(NRC):
            sl = pl.ds(r0 + t * RCHUNK, RCHUNK)
            pltpu.sync_copy(acc.at[sl], out_hbm.at[c].at[sl])

        pltpu.sync_copy(dacc.at[pl.ds(r0, RPT)],
                        deg_hbm.at[c].at[pl.ds(r0, RPT)])

    return k(xw_flat, packed_idx)


def _finish_body(a_ref, d0_ref, d1_ref, b_ref, o_ref):
    d = jnp.maximum(d0_ref[...] + d1_ref[...], 1.0)
    o_ref[:, :DH] = a_ref[0] / d + b_ref[0, :DH]
    o_ref[:, DH:] = a_ref[1] / d + b_ref[0, DH:]


def _finish(acc2, dega, degb, b2):
    return pl.pallas_call(
        _finish_body,
        grid=(N // ROWB,),
        in_specs=[pl.BlockSpec((NCORE, ROWB, DH), lambda r: (0, r, 0)),
                  pl.BlockSpec((ROWB, 1), lambda r: (r, 0)),
                  pl.BlockSpec((ROWB, 1), lambda r: (r, 0)),
                  pl.BlockSpec((1, D), lambda r: (0, 0))],
        out_specs=pl.BlockSpec((ROWB, D), lambda r: (r, 0)),
        out_shape=jax.ShapeDtypeStruct((N, D), jnp.float32),
    )(acc2, dega, degb, b2)


def kernel(x, edge_index, W, b):
    e4 = edge_index.astype(jnp.int32).reshape(2, NSUB, 1, EPT)
    xw, packed = _xw_split(x, W, e4)
    xw_flat = xw.reshape(NCORE * N, DH)
    acc, deg = _sc_scatter(
        xw_flat, packed.reshape(NSUB, NCORE, NCHUNK, K))
    return _finish(acc, deg[0, :N].reshape(N, 1), deg[1, :N].reshape(N, 1),
                   b.reshape(1, D))
